# Initial kernel scaffold; baseline (speedup 1.0000x reference)
import jax
import jax.numpy as jnp
from jax.experimental import pallas as pl


def kernel(x):
    raise NotImplementedError("write your pallas kernel here")


# speedup vs baseline: 21.3682x; 21.3682x over previous
import jax
import jax.numpy as jnp
from jax.experimental import pallas as pl

_TOPK = 256


def _topk_mask_kernel(x_ref, o_ref):
    x = x_ref[...]
    u = jax.lax.bitcast_convert_type(x, jnp.uint32)
    s = u >> jnp.uint32(31)
    flip = jnp.where(s == 0, jnp.uint32(0x80000000), jnp.uint32(0xFFFFFFFF))
    m = u ^ flip

    def body(i, prefix):
        b = (jnp.uint32(31) - i.astype(jnp.uint32))
        cand = prefix | jnp.left_shift(jnp.uint32(1), b)
        cnt = jnp.sum((m >= cand[:, None]).astype(jnp.int32), axis=1)
        return jnp.where(cnt >= _TOPK, cand, prefix)

    prefix0 = jnp.zeros((x.shape[0],), jnp.uint32)
    thr = jax.lax.fori_loop(0, 32, body, prefix0)

    R, C = x.shape
    gt = m > thr[:, None]
    eq_f = (m == thr[:, None]).astype(jnp.float32)
    need = (jnp.float32(_TOPK)
            - jnp.sum(gt.astype(jnp.float32), axis=1))

    nchunk = C // 128
    e3 = eq_f.reshape(R * nchunk, 128)
    tri128 = (jax.lax.broadcasted_iota(jnp.int32, (128, 128), 0)
              <= jax.lax.broadcasted_iota(jnp.int32, (128, 128), 1)
              ).astype(jnp.float32)
    pref_in = jnp.dot(e3, tri128,
                      preferred_element_type=jnp.float32)
    pref_in = pref_in.reshape(R, nchunk, 128)
    chunk_tot = eq_f.reshape(R, nchunk, 128).sum(axis=2)
    trin = (jax.lax.broadcasted_iota(jnp.int32, (nchunk, nchunk), 0)
            < jax.lax.broadcasted_iota(jnp.int32, (nchunk, nchunk), 1)
            ).astype(jnp.float32)
    chunk_excl = jnp.dot(chunk_tot, trin,
                         preferred_element_type=jnp.float32)
    rank = (pref_in + chunk_excl[:, :, None]).reshape(R, C)
    keep_eq = (eq_f > 0) & (rank <= need[:, None])
    o_ref[...] = jnp.where(gt | keep_eq, x, jnp.float32(0.0))


@jax.jit
def kernel(x):
    R, C = x.shape
    BR = 16
    return pl.pallas_call(
        _topk_mask_kernel,
        grid=(R // BR,),
        in_specs=[pl.BlockSpec((BR, C), lambda i: (i, 0))],
        out_specs=pl.BlockSpec((BR, C), lambda i: (i, 0)),
        out_shape=jax.ShapeDtypeStruct((R, C), x.dtype),
    )(x)
